# Initial kernel scaffold; baseline (speedup 1.0000x reference)
import jax
import jax.numpy as jnp
from jax.experimental import pallas as pl


def kernel(batch, lens, table, W1, b1, W2, b2, W3, b3):
    raise NotImplementedError("write your pallas kernel here")


# speedup vs baseline: 1.3013x; 1.3013x over previous
import jax
import jax.numpy as jnp
from jax import lax
from jax.experimental import pallas as pl
from jax.experimental.pallas import tpu as pltpu
from jax.experimental.pallas import tpu_sc as plsc

B, L, V, D = 4096, 50, 100000, 300
DP = 304
NC, NS = 2, 16
NW = NC * NS
BPW = B // NW
LANES = 16
NCH = DP // LANES


def _pool_body(batch_hbm, table_hbm, pooled_hbm, idx_v, rows0, rows1, out_v,
               sem0, sem1):
    wid = lax.axis_index("s") * NC + lax.axis_index("c")
    base = wid * BPW
    pltpu.sync_copy(batch_hbm.at[pl.ds(base, BPW)], idx_v)

    bufs = ((rows0, sem0), (rows1, sem1))

    def gather(e, buf, sem):
        return pltpu.make_async_copy(table_hbm.at[idx_v.at[e]], buf, sem)

    gather(0, rows0, sem0).start()
    gather(1, rows1, sem1).start()

    zero = jnp.zeros((LANES,), jnp.float32)
    init = tuple(zero for _ in range(NCH))

    def accumulate(e, buf):
        def rbody(r, acc):
            return tuple(acc[j] + buf[r, pl.ds(LANES * j, LANES)]
                         for j in range(NCH))
        acc = lax.fori_loop(0, L, rbody, init)
        for j in range(NCH):
            out_v[e, pl.ds(LANES * j, LANES)] = acc[j]

    def pair(i, carry):
        e0 = i * 2
        for b in range(2):
            buf, sem = bufs[b]
            e = e0 + b
            gather(e, buf, sem).wait()
            accumulate(e, buf)
            nxt = e + 2

            @pl.when(nxt < BPW)
            def _():
                gather(nxt, buf, sem).start()
        return carry

    lax.fori_loop(0, BPW // 2, pair, 0)
    pltpu.sync_copy(out_v, pooled_hbm.at[pl.ds(base, BPW)])


def _pool(batch, table_p):
    mesh = plsc.VectorSubcoreMesh(core_axis_name="c", subcore_axis_name="s")
    k = pl.kernel(
        _pool_body,
        mesh=mesh,
        compiler_params=pltpu.CompilerParams(use_tc_tiling_on_sc=False),
        out_type=jax.ShapeDtypeStruct((B, DP), jnp.float32),
        scratch_types=[
            pltpu.VMEM((BPW, L), jnp.int32),
            pltpu.VMEM((L, DP), jnp.float32),
            pltpu.VMEM((L, DP), jnp.float32),
            pltpu.VMEM((BPW, DP), jnp.float32),
            pltpu.SemaphoreType.DMA,
            pltpu.SemaphoreType.DMA,
        ],
    )
    return k(batch, table_p)


def _mlp_body(x_ref, lens_ref, w1_ref, b1_ref, w2_ref, b2_ref, w3_ref, b3_ref,
              o_ref):
    x = x_ref[...] / lens_ref[...].astype(jnp.float32)
    cdims = (((1,), (1,)), ((), ()))
    h1 = lax.dot_general(x, w1_ref[...], cdims,
                         preferred_element_type=jnp.float32)
    h1 = jnp.maximum(h1 + b1_ref[...], 0.0)
    h2 = lax.dot_general(h1, w2_ref[...], cdims,
                         preferred_element_type=jnp.float32)
    h2 = jnp.maximum(h2 + b2_ref[...], 0.0)
    h3 = jnp.sum(h2 * w3_ref[...], axis=1, keepdims=True)
    o_ref[...] = h3 + b3_ref[0, 0]


def _mlp(pooled, lens, W1p, b1, W2, b2, W3, b3):
    BB = 512
    grid = (B // BB,)
    return pl.pallas_call(
        _mlp_body,
        grid=grid,
        in_specs=[
            pl.BlockSpec((BB, DP), lambda i: (i, 0)),
            pl.BlockSpec((BB, 1), lambda i: (i, 0)),
            pl.BlockSpec((150, DP), lambda i: (0, 0)),
            pl.BlockSpec((1, 150), lambda i: (0, 0)),
            pl.BlockSpec((150, 150), lambda i: (0, 0)),
            pl.BlockSpec((1, 150), lambda i: (0, 0)),
            pl.BlockSpec((1, 150), lambda i: (0, 0)),
            pl.BlockSpec(memory_space=pltpu.MemorySpace.SMEM),
        ],
        out_specs=pl.BlockSpec((BB, 1), lambda i: (i, 0)),
        out_shape=jax.ShapeDtypeStruct((B, 1), jnp.float32),
    )(pooled, lens, W1p, b1, W2, b2, W3, b3)


def kernel(batch, lens, table, W1, b1, W2, b2, W3, b3):
    table_p = jnp.pad(table, ((0, 0), (0, DP - D)))
    W1p = jnp.pad(W1, ((0, 0), (0, DP - D)))
    pooled = _pool(batch, table_p)
    lens2 = lens.reshape(B, 1)
    out = _mlp(pooled, lens2, W1p, b1.reshape(1, 150), W2, b2.reshape(1, 150),
               W3, b3.reshape(1, 1))
    return out.reshape((B,))


# speedup vs baseline: 1.5620x; 1.2003x over previous
import jax
import jax.numpy as jnp
from jax import lax
from jax.experimental import pallas as pl
from jax.experimental.pallas import tpu as pltpu
from jax.experimental.pallas import tpu_sc as plsc

B, L, V, D = 4096, 50, 100000, 300
DP = 384
NC, NS = 2, 16
NW = NC * NS
BPW = B // NW
LANES = 16
NCH = DP // LANES


def _pool_body(batch_hbm, table_hbm, pooled_hbm, idx_v, rows0, rows1, out_v,
               sem0, sem1):
    wid = lax.axis_index("s") * NC + lax.axis_index("c")
    base = wid * BPW
    pltpu.sync_copy(batch_hbm.at[pl.ds(base, BPW)], idx_v)

    bufs = ((rows0, sem0), (rows1, sem1))

    def gather_copies(e, buf, sem):
        return [
            pltpu.make_async_copy(
                table_hbm.at[idx_v.at[e], pl.ds(ct * 128, 128)],
                buf.at[:, pl.ds(ct * 128, 128)], sem)
            for ct in range(DP // 128)
        ]

    def gather_start(e, buf, sem):
        for c in gather_copies(e, buf, sem):
            c.start()

    def gather_wait(e, buf, sem):
        for c in gather_copies(e, buf, sem):
            c.wait()

    gather_start(0, rows0, sem0)
    gather_start(1, rows1, sem1)

    zero = jnp.zeros((LANES,), jnp.float32)
    init = tuple(zero for _ in range(NCH))

    def accumulate(e, buf):
        def rbody(r, acc):
            return tuple(acc[j] + buf[r, pl.ds(LANES * j, LANES)]
                         for j in range(NCH))
        acc = lax.fori_loop(0, L, rbody, init)
        for j in range(NCH):
            out_v[e, pl.ds(LANES * j, LANES)] = acc[j]

    def pair(i, carry):
        e0 = i * 2
        for b in range(2):
            buf, sem = bufs[b]
            e = e0 + b
            gather_wait(e, buf, sem)
            accumulate(e, buf)
            nxt = e + 2

            @pl.when(nxt < BPW)
            def _():
                gather_start(nxt, buf, sem)
        return carry

    lax.fori_loop(0, BPW // 2, pair, 0)
    pltpu.sync_copy(out_v, pooled_hbm.at[pl.ds(base, BPW)])


def _pool(batch, table_p):
    mesh = plsc.VectorSubcoreMesh(core_axis_name="c", subcore_axis_name="s")
    k = pl.kernel(
        _pool_body,
        mesh=mesh,
        out_type=jax.ShapeDtypeStruct((B, DP), jnp.float32),
        scratch_types=[
            pltpu.VMEM((BPW, L), jnp.int32),
            pltpu.VMEM((L, DP), jnp.float32),
            pltpu.VMEM((L, DP), jnp.float32),
            pltpu.VMEM((BPW, DP), jnp.float32),
            pltpu.SemaphoreType.DMA,
            pltpu.SemaphoreType.DMA,
        ],
    )
    return k(batch, table_p)


def _mlp_body(x_ref, lens_ref, w1_ref, b1_ref, w2_ref, b2_ref, w3_ref, b3_ref,
              o_ref):
    x = x_ref[...] / lens_ref[...].astype(jnp.float32)
    cdims = (((1,), (1,)), ((), ()))
    h1 = lax.dot_general(x, w1_ref[...], cdims,
                         preferred_element_type=jnp.float32)
    h1 = jnp.maximum(h1 + b1_ref[...], 0.0)
    h2 = lax.dot_general(h1, w2_ref[...], cdims,
                         preferred_element_type=jnp.float32)
    h2 = jnp.maximum(h2 + b2_ref[...], 0.0)
    h3 = jnp.sum(h2 * w3_ref[...], axis=1, keepdims=True)
    o_ref[...] = h3 + b3_ref[0, 0]


def _mlp(pooled, lens, W1p, b1, W2, b2, W3, b3):
    BB = 512
    grid = (B // BB,)
    return pl.pallas_call(
        _mlp_body,
        grid=grid,
        in_specs=[
            pl.BlockSpec((BB, DP), lambda i: (i, 0)),
            pl.BlockSpec((BB, 1), lambda i: (i, 0)),
            pl.BlockSpec((150, DP), lambda i: (0, 0)),
            pl.BlockSpec((1, 150), lambda i: (0, 0)),
            pl.BlockSpec((150, 150), lambda i: (0, 0)),
            pl.BlockSpec((1, 150), lambda i: (0, 0)),
            pl.BlockSpec((1, 150), lambda i: (0, 0)),
            pl.BlockSpec(memory_space=pltpu.MemorySpace.SMEM),
        ],
        out_specs=pl.BlockSpec((BB, 1), lambda i: (i, 0)),
        out_shape=jax.ShapeDtypeStruct((B, 1), jnp.float32),
    )(pooled, lens, W1p, b1, W2, b2, W3, b3)


def kernel(batch, lens, table, W1, b1, W2, b2, W3, b3):
    table_p = jnp.pad(table, ((0, 0), (0, DP - D)))
    W1p = jnp.pad(W1, ((0, 0), (0, DP - D)))
    pooled = _pool(batch, table_p)
    lens2 = lens.reshape(B, 1)
    out = _mlp(pooled, lens2, W1p, b1.reshape(1, 150), W2, b2.reshape(1, 150),
               W3, b3.reshape(1, 1))
    return out.reshape((B,))


# speedup vs baseline: 2.6866x; 1.7200x over previous
import jax
import jax.numpy as jnp
from jax import lax
from jax.experimental import pallas as pl
from jax.experimental.pallas import tpu as pltpu
from jax.experimental.pallas import tpu_sc as plsc

B, L, V, D = 4096, 50, 100000, 300
DA = 256
DT = 48
NC, NS = 2, 16
NW = NC * NS
BPW = B // NW
LANES = 16


def _pool_body(batch_hbm, table_hbm, pooled_hbm, idx_v, rows0, rows1, out_v,
               sem0, sem1, *, width, tiles):
    nch = width // LANES
    wid = lax.axis_index("s") * NC + lax.axis_index("c")
    base = wid * BPW
    pltpu.sync_copy(batch_hbm.at[pl.ds(base, BPW)], idx_v)

    bufs = ((rows0, sem0), (rows1, sem1))

    def gather_copies(e, buf, sem):
        if tiles == 1:
            return [pltpu.make_async_copy(table_hbm.at[idx_v.at[e]], buf,
                                          sem)]
        return [
            pltpu.make_async_copy(
                table_hbm.at[idx_v.at[e], pl.ds(ct * 128, 128)],
                buf.at[:, pl.ds(ct * 128, 128)], sem)
            for ct in range(tiles)
        ]

    def gather_start(e, buf, sem):
        for c in gather_copies(e, buf, sem):
            c.start()

    def gather_wait(e, buf, sem):
        for c in gather_copies(e, buf, sem):
            c.wait()

    gather_start(0, rows0, sem0)
    gather_start(1, rows1, sem1)

    zero = jnp.zeros((LANES,), jnp.float32)
    init = tuple(zero for _ in range(nch))

    def accumulate(e, buf):
        def rbody(r, acc):
            return tuple(acc[j] + buf[r, pl.ds(LANES * j, LANES)]
                         for j in range(nch))
        acc = lax.fori_loop(0, L, rbody, init)
        for j in range(nch):
            out_v[e, pl.ds(LANES * j, LANES)] = acc[j]

    def pair(i, carry):
        e0 = i * 2
        for b in range(2):
            buf, sem = bufs[b]
            e = e0 + b
            gather_wait(e, buf, sem)
            accumulate(e, buf)
            nxt = e + 2

            @pl.when(nxt < BPW)
            def _():
                gather_start(nxt, buf, sem)
        return carry

    lax.fori_loop(0, BPW // 2, pair, 0)
    pltpu.sync_copy(out_v, pooled_hbm.at[pl.ds(base, BPW)])


def _make_pool(width, tiles, use_tc_tiling):
    import functools
    mesh = plsc.VectorSubcoreMesh(core_axis_name="c", subcore_axis_name="s")
    body = functools.partial(_pool_body, width=width, tiles=tiles)
    kw = {}
    if not use_tc_tiling:
        kw["compiler_params"] = pltpu.CompilerParams(use_tc_tiling_on_sc=False)
    return pl.kernel(
        body,
        mesh=mesh,
        out_type=jax.ShapeDtypeStruct((B, width), jnp.float32),
        scratch_types=[
            pltpu.VMEM((BPW, L), jnp.int32),
            pltpu.VMEM((L, width), jnp.float32),
            pltpu.VMEM((L, width), jnp.float32),
            pltpu.VMEM((BPW, width), jnp.float32),
            pltpu.SemaphoreType.DMA,
            pltpu.SemaphoreType.DMA,
        ],
        **kw,
    )


def _mlp_body(xa_ref, xb_ref, lens_ref, w1a_ref, w1b_ref, b1_ref, w2_ref,
              b2_ref, w3_ref, b3_ref, o_ref):
    recip = 1.0 / lens_ref[...].astype(jnp.float32)
    xa = xa_ref[...] * recip
    xb = xb_ref[...] * recip
    cdims = (((1,), (1,)), ((), ()))
    h1 = (lax.dot_general(xa, w1a_ref[...], cdims,
                          preferred_element_type=jnp.float32)
          + lax.dot_general(xb, w1b_ref[...], cdims,
                            preferred_element_type=jnp.float32))
    h1 = jnp.maximum(h1 + b1_ref[...], 0.0)
    h2 = lax.dot_general(h1, w2_ref[...], cdims,
                         preferred_element_type=jnp.float32)
    h2 = jnp.maximum(h2 + b2_ref[...], 0.0)
    h3 = jnp.sum(h2 * w3_ref[...], axis=1, keepdims=True)
    o_ref[...] = h3 + b3_ref[0, 0]


def _mlp(pooled_a, pooled_b, lens, W1a, W1b, b1, W2, b2, W3, b3):
    BB = 512
    grid = (B // BB,)
    return pl.pallas_call(
        _mlp_body,
        grid=grid,
        in_specs=[
            pl.BlockSpec((BB, DA), lambda i: (i, 0)),
            pl.BlockSpec((BB, DT), lambda i: (i, 0)),
            pl.BlockSpec((BB, 1), lambda i: (i, 0)),
            pl.BlockSpec((150, DA), lambda i: (0, 0)),
            pl.BlockSpec((150, DT), lambda i: (0, 0)),
            pl.BlockSpec((1, 150), lambda i: (0, 0)),
            pl.BlockSpec((150, 150), lambda i: (0, 0)),
            pl.BlockSpec((1, 150), lambda i: (0, 0)),
            pl.BlockSpec((1, 150), lambda i: (0, 0)),
            pl.BlockSpec(memory_space=pltpu.MemorySpace.SMEM),
        ],
        out_specs=pl.BlockSpec((BB, 1), lambda i: (i, 0)),
        out_shape=jax.ShapeDtypeStruct((B, 1), jnp.float32),
    )(pooled_a, pooled_b, lens, W1a, W1b, b1, W2, b2, W3, b3)


def kernel(batch, lens, table, W1, b1, W2, b2, W3, b3):
    table_tail = jnp.pad(table[:, DA:], ((0, 0), (0, DT - (D - DA))))
    pooled_a = _make_pool(DA, 2, True)(batch, table)
    pooled_b = _make_pool(DT, 1, False)(batch, table_tail)
    W1a = W1[:, :DA]
    W1b = jnp.pad(W1[:, DA:], ((0, 0), (0, DT - (D - DA))))
    lens2 = lens.reshape(B, 1)
    out = _mlp(pooled_a, pooled_b, lens2, W1a, W1b, b1.reshape(1, 150), W2,
               b2.reshape(1, 150), W3, b3.reshape(1, 1))
    return out.reshape((B,))


# speedup vs baseline: 3.1487x; 1.1720x over previous
import functools

import jax
import jax.numpy as jnp
from jax import lax
from jax.experimental import pallas as pl
from jax.experimental.pallas import tpu as pltpu
from jax.experimental.pallas import tpu_sc as plsc

B, L, V, D = 4096, 50, 100000, 300
DA = 256
TAIL0 = D - 128
DT = 48
NC, NS = 2, 16
NW = NC * NS
BPW = B // NW
LANES = 16


def _pool_a_body(batch_hbm, table_hbm, pooled_hbm, idx_v, rows0, rows1, out_v,
                 sem0, sem1):
    nch = DA // LANES
    wid = lax.axis_index("s") * NC + lax.axis_index("c")
    base = wid * BPW
    pltpu.sync_copy(batch_hbm.at[pl.ds(base, BPW)], idx_v)

    bufs = ((rows0, sem0), (rows1, sem1))

    def gather_copies(e, buf, sem):
        return [
            pltpu.make_async_copy(
                table_hbm.at[idx_v.at[e], pl.ds(ct * 128, 128)],
                buf.at[:, pl.ds(ct * 128, 128)], sem)
            for ct in range(DA // 128)
        ]

    def gather_start(e, buf, sem):
        for c in gather_copies(e, buf, sem):
            c.start()

    def gather_wait(e, buf, sem):
        for c in gather_copies(e, buf, sem):
            c.wait()

    gather_start(0, rows0, sem0)
    gather_start(1, rows1, sem1)

    zero = jnp.zeros((LANES,), jnp.float32)
    init = tuple(zero for _ in range(nch))

    def accumulate(e, buf):
        def rbody(r, acc):
            return tuple(acc[j] + buf[r, pl.ds(LANES * j, LANES)]
                         for j in range(nch))
        acc = lax.fori_loop(0, L, rbody, init)
        for j in range(nch):
            out_v[e, pl.ds(LANES * j, LANES)] = acc[j]

    def pair(i, carry):
        e0 = i * 2
        for b in range(2):
            buf, sem = bufs[b]
            e = e0 + b
            gather_wait(e, buf, sem)
            accumulate(e, buf)
            nxt = e + 2

            @pl.when(nxt < BPW)
            def _():
                gather_start(nxt, buf, sem)
        return carry

    lax.fori_loop(0, BPW // 2, pair, 0)
    pltpu.sync_copy(out_v, pooled_hbm.at[pl.ds(base, BPW)])


def _pool_b_body(batch_hbm, tail_hbm, pooled_hbm, idx_v, rows0, rows1, out_v,
                 sem0, sem1):
    offs = (80, 96, 112)
    wid = lax.axis_index("s") * NC + lax.axis_index("c")
    base = wid * BPW
    pltpu.sync_copy(batch_hbm.at[pl.ds(base, BPW)], idx_v)

    bufs = ((rows0, sem0), (rows1, sem1))

    def gather(e, buf, sem):
        return pltpu.make_async_copy(tail_hbm.at[idx_v.at[e]], buf, sem)

    gather(0, rows0, sem0).start()
    gather(1, rows1, sem1).start()

    zero = jnp.zeros((LANES,), jnp.float32)
    init = (zero, zero, zero)
    lane_ge4 = lax.iota(jnp.int32, LANES) >= 4

    def accumulate(e, buf):
        def rbody(r, acc):
            return tuple(acc[j] + buf[r, pl.ds(offs[j], LANES)]
                         for j in range(3))
        acc = lax.fori_loop(0, L, rbody, init)
        out_v[e, pl.ds(0, LANES)] = jnp.where(lane_ge4, acc[0], 0.0)
        out_v[e, pl.ds(16, LANES)] = acc[1]
        out_v[e, pl.ds(32, LANES)] = acc[2]

    def pair(i, carry):
        e0 = i * 2
        for b in range(2):
            buf, sem = bufs[b]
            e = e0 + b
            gather(e, buf, sem).wait()
            accumulate(e, buf)
            nxt = e + 2

            @pl.when(nxt < BPW)
            def _():
                gather(nxt, buf, sem).start()
        return carry

    lax.fori_loop(0, BPW // 2, pair, 0)
    pltpu.sync_copy(out_v, pooled_hbm.at[pl.ds(base, BPW)])


def _make_pool(body, width):
    mesh = plsc.VectorSubcoreMesh(core_axis_name="c", subcore_axis_name="s")
    return pl.kernel(
        body,
        mesh=mesh,
        out_type=jax.ShapeDtypeStruct((B, DT if body is _pool_b_body else width),
                                      jnp.float32),
        scratch_types=[
            pltpu.VMEM((BPW, L), jnp.int32),
            pltpu.VMEM((L, width), jnp.float32),
            pltpu.VMEM((L, width), jnp.float32),
            pltpu.VMEM((BPW, DT if body is _pool_b_body else width),
                       jnp.float32),
            pltpu.SemaphoreType.DMA,
            pltpu.SemaphoreType.DMA,
        ],
    )


def _mlp_body(xa_ref, xb_ref, lens_ref, w1a_ref, w1b_ref, b1_ref, w2_ref,
              b2_ref, w3_ref, b3_ref, o_ref):
    recip = 1.0 / lens_ref[...].astype(jnp.float32)
    xa = xa_ref[...] * recip
    xb = xb_ref[...] * recip
    cdims = (((1,), (1,)), ((), ()))
    h1 = (lax.dot_general(xa, w1a_ref[...], cdims,
                          preferred_element_type=jnp.float32)
          + lax.dot_general(xb, w1b_ref[...], cdims,
                            preferred_element_type=jnp.float32))
    h1 = jnp.maximum(h1 + b1_ref[...], 0.0)
    h2 = lax.dot_general(h1, w2_ref[...], cdims,
                         preferred_element_type=jnp.float32)
    h2 = jnp.maximum(h2 + b2_ref[...], 0.0)
    h3 = jnp.sum(h2 * w3_ref[...], axis=1, keepdims=True)
    o_ref[...] = h3 + b3_ref[0, 0]


def _mlp(pooled_a, pooled_b, lens, W1a, W1b, b1, W2, b2, W3, b3):
    BB = 512
    grid = (B // BB,)
    return pl.pallas_call(
        _mlp_body,
        grid=grid,
        in_specs=[
            pl.BlockSpec((BB, DA), lambda i: (i, 0)),
            pl.BlockSpec((BB, DT), lambda i: (i, 0)),
            pl.BlockSpec((BB, 1), lambda i: (i, 0)),
            pl.BlockSpec((150, DA), lambda i: (0, 0)),
            pl.BlockSpec((150, DT), lambda i: (0, 0)),
            pl.BlockSpec((1, 150), lambda i: (0, 0)),
            pl.BlockSpec((150, 150), lambda i: (0, 0)),
            pl.BlockSpec((1, 150), lambda i: (0, 0)),
            pl.BlockSpec((1, 150), lambda i: (0, 0)),
            pl.BlockSpec(memory_space=pltpu.MemorySpace.SMEM),
        ],
        out_specs=pl.BlockSpec((BB, 1), lambda i: (i, 0)),
        out_shape=jax.ShapeDtypeStruct((B, 1), jnp.float32),
    )(pooled_a, pooled_b, lens, W1a, W1b, b1, W2, b2, W3, b3)


def kernel(batch, lens, table, W1, b1, W2, b2, W3, b3):
    tail_table = table[:, TAIL0:]
    pooled_a = _make_pool(_pool_a_body, DA)(batch, table)
    tail2, pooled_a = lax.optimization_barrier((tail_table, pooled_a))
    pooled_b = _make_pool(_pool_b_body, 128)(batch, tail2)
    W1b = jnp.pad(W1[:, DA:], ((0, 0), (4, 0)))
    W1a = W1[:, :DA]
    lens2 = lens.reshape(B, 1)
    out = _mlp(pooled_a, pooled_b, lens2, W1a, W1b, b1.reshape(1, 150), W2,
               b2.reshape(1, 150), W3, b3.reshape(1, 1))
    return out.reshape((B,))


# speedup vs baseline: 3.2844x; 1.0431x over previous
import jax
import jax.numpy as jnp
from jax import lax
from jax.experimental import pallas as pl
from jax.experimental.pallas import tpu as pltpu
from jax.experimental.pallas import tpu_sc as plsc

B, L, V, D = 4096, 50, 100000, 300
DA = 256
TAIL0 = D - 128
DT = 48
NC, NS = 2, 16
NW = NC * NS
BPW = B // NW
LANES = 16


def _pool_a_body(batcht_hbm, table_hbm, pooled_hbm, idx_s, rows0, rows1,
                 out_v, sem0, sem1):
    nch = DA // LANES
    wid = lax.axis_index("s") * NC + lax.axis_index("c")
    base = wid * BPW
    pltpu.sync_copy(batcht_hbm.at[:, pl.ds(base, BPW)], idx_s)

    bufs = ((rows0, sem0), (rows1, sem1))

    def gather_copies(r, buf, sem):
        return [
            pltpu.make_async_copy(
                table_hbm.at[idx_s.at[r], pl.ds(ct * 128, 128)],
                buf.at[:, pl.ds(ct * 128, 128)], sem)
            for ct in range(DA // 128)
        ]

    def gather_start(r, buf, sem):
        for c in gather_copies(r, buf, sem):
            c.start()

    def gather_wait(r, buf, sem):
        for c in gather_copies(r, buf, sem):
            c.wait()

    gather_start(0, rows0, sem0)
    gather_start(1, rows1, sem1)

    zero = jnp.zeros((LANES,), jnp.float32)

    def zbody(e, carry):
        for j in range(nch):
            out_v[e, pl.ds(LANES * j, LANES)] = zero
        return carry

    lax.fori_loop(0, BPW, zbody, 0)

    def accumulate(buf):
        def ebody(e, carry):
            for j in range(nch):
                plsc.addupdate(out_v.at[e, pl.ds(LANES * j, LANES)],
                               buf[e, pl.ds(LANES * j, LANES)])
            return carry
        lax.fori_loop(0, BPW, ebody, 0)

    def pair(i, carry):
        r0 = i * 2
        for b in range(2):
            buf, sem = bufs[b]
            r = r0 + b
            gather_wait(r, buf, sem)
            accumulate(buf)
            nxt = r + 2

            @pl.when(nxt < L)
            def _():
                gather_start(nxt, buf, sem)
        return carry

    lax.fori_loop(0, L // 2, pair, 0)
    pltpu.sync_copy(out_v, pooled_hbm.at[pl.ds(base, BPW)])


def _pool_b_body(batcht_hbm, tail_hbm, pooled_hbm, idx_s, rows0, rows1,
                 out_v, sem0, sem1):
    offs = (80, 96, 112)
    wid = lax.axis_index("s") * NC + lax.axis_index("c")
    base = wid * BPW
    pltpu.sync_copy(batcht_hbm.at[:, pl.ds(base, BPW)], idx_s)

    bufs = ((rows0, sem0), (rows1, sem1))

    def gather(r, buf, sem):
        return pltpu.make_async_copy(tail_hbm.at[idx_s.at[r]], buf, sem)

    gather(0, rows0, sem0).start()
    gather(1, rows1, sem1).start()

    zero = jnp.zeros((LANES,), jnp.float32)

    def zbody(e, carry):
        for j in range(3):
            out_v[e, pl.ds(LANES * j, LANES)] = zero
        return carry

    lax.fori_loop(0, BPW, zbody, 0)

    def accumulate(buf):
        def ebody(e, carry):
            for j in range(3):
                plsc.addupdate(out_v.at[e, pl.ds(LANES * j, LANES)],
                               buf[e, pl.ds(offs[j], LANES)])
            return carry
        lax.fori_loop(0, BPW, ebody, 0)

    def pair(i, carry):
        r0 = i * 2
        for b in range(2):
            buf, sem = bufs[b]
            r = r0 + b
            gather(r, buf, sem).wait()
            accumulate(buf)
            nxt = r + 2

            @pl.when(nxt < L)
            def _():
                gather(nxt, buf, sem).start()
        return carry

    lax.fori_loop(0, L // 2, pair, 0)

    lane_ge4 = lax.iota(jnp.int32, LANES) >= 4

    def mbody(e, carry):
        out_v[e, pl.ds(0, LANES)] = jnp.where(
            lane_ge4, out_v[e, pl.ds(0, LANES)], 0.0)
        return carry

    lax.fori_loop(0, BPW, mbody, 0)
    pltpu.sync_copy(out_v, pooled_hbm.at[pl.ds(base, BPW)])


def _make_pool(body, width, out_w):
    mesh = plsc.VectorSubcoreMesh(core_axis_name="c", subcore_axis_name="s")
    return pl.kernel(
        body,
        mesh=mesh,
        out_type=jax.ShapeDtypeStruct((B, out_w), jnp.float32),
        scratch_types=[
            pltpu.VMEM((L, BPW), jnp.int32),
            pltpu.VMEM((BPW, width), jnp.float32),
            pltpu.VMEM((BPW, width), jnp.float32),
            pltpu.VMEM((BPW, out_w), jnp.float32),
            pltpu.SemaphoreType.DMA,
            pltpu.SemaphoreType.DMA,
        ],
    )


def _mlp_body(xa_ref, xb_ref, lens_ref, w1a_ref, w1b_ref, b1_ref, w2_ref,
              b2_ref, w3_ref, b3_ref, o_ref):
    recip = 1.0 / lens_ref[...].astype(jnp.float32)
    xa = xa_ref[...] * recip
    xb = xb_ref[...] * recip
    cdims = (((1,), (1,)), ((), ()))
    h1 = (lax.dot_general(xa, w1a_ref[...], cdims,
                          preferred_element_type=jnp.float32)
          + lax.dot_general(xb, w1b_ref[...], cdims,
                            preferred_element_type=jnp.float32))
    h1 = jnp.maximum(h1 + b1_ref[...], 0.0)
    h2 = lax.dot_general(h1, w2_ref[...], cdims,
                         preferred_element_type=jnp.float32)
    h2 = jnp.maximum(h2 + b2_ref[...], 0.0)
    h3 = jnp.sum(h2 * w3_ref[...], axis=1, keepdims=True)
    o_ref[...] = h3 + b3_ref[0, 0]


def _mlp(pooled_a, pooled_b, lens, W1a, W1b, b1, W2, b2, W3, b3):
    BB = 512
    grid = (B // BB,)
    return pl.pallas_call(
        _mlp_body,
        grid=grid,
        in_specs=[
            pl.BlockSpec((BB, DA), lambda i: (i, 0)),
            pl.BlockSpec((BB, DT), lambda i: (i, 0)),
            pl.BlockSpec((BB, 1), lambda i: (i, 0)),
            pl.BlockSpec((150, DA), lambda i: (0, 0)),
            pl.BlockSpec((150, DT), lambda i: (0, 0)),
            pl.BlockSpec((1, 150), lambda i: (0, 0)),
            pl.BlockSpec((150, 150), lambda i: (0, 0)),
            pl.BlockSpec((1, 150), lambda i: (0, 0)),
            pl.BlockSpec((1, 150), lambda i: (0, 0)),
            pl.BlockSpec(memory_space=pltpu.MemorySpace.SMEM),
        ],
        out_specs=pl.BlockSpec((BB, 1), lambda i: (i, 0)),
        out_shape=jax.ShapeDtypeStruct((B, 1), jnp.float32),
    )(pooled_a, pooled_b, lens, W1a, W1b, b1, W2, b2, W3, b3)


def kernel(batch, lens, table, W1, b1, W2, b2, W3, b3):
    batcht = batch.T
    tail_table = table[:, TAIL0:]
    pooled_a = _make_pool(_pool_a_body, DA, DA)(batcht, table)
    tail2, pooled_a = lax.optimization_barrier((tail_table, pooled_a))
    pooled_b = _make_pool(_pool_b_body, 128, DT)(batcht, tail2)
    W1b = jnp.pad(W1[:, DA:], ((0, 0), (4, 0)))
    W1a = W1[:, :DA]
    lens2 = lens.reshape(B, 1)
    out = _mlp(pooled_a, pooled_b, lens2, W1a, W1b, b1.reshape(1, 150), W2,
               b2.reshape(1, 150), W3, b3.reshape(1, 1))
    return out.reshape((B,))
